# Initial kernel scaffold; baseline (speedup 1.0000x reference)
import jax
import jax.numpy as jnp
from jax.experimental import pallas as pl


def kernel(x, edge_index, batch, w1a, b1a, w1b, b1b, w2a, b2a, w2b, b2b, w3a, b3a, w3b, b3b, wl1, bl1, wl2, bl2):
    raise NotImplementedError("write your pallas kernel here")


# speedup vs baseline: 1.0213x; 1.0213x over previous
import functools

import jax
import jax.numpy as jnp
from jax import lax
from jax.experimental import pallas as pl
from jax.experimental.pallas import tpu as pltpu
from jax.experimental.pallas import tpu_sc as plsc

_BLK = 512


def _mlp_layer_body(z_ref, wa_ref, ba_ref, wb_ref, bb_ref, out_ref):
    z = z_ref[...]
    t = jnp.maximum(
        jnp.dot(z, wa_ref[...], preferred_element_type=jnp.float32) + ba_ref[...], 0.0)
    o = jnp.maximum(
        jnp.dot(t, wb_ref[...], preferred_element_type=jnp.float32) + bb_ref[...], 0.0)
    out_ref[...] = o


def _mlp_layer(z, wa, ba, wb, bb, interpret=False):
    npad, din = z.shape
    dout = wb.shape[1]
    nb = npad // _BLK
    return pl.pallas_call(
        _mlp_layer_body,
        grid=(nb,),
        in_specs=[
            pl.BlockSpec((_BLK, din), lambda i: (i, 0)),
            pl.BlockSpec((din, 512), lambda i: (0, 0)),
            pl.BlockSpec((1, 512), lambda i: (0, 0)),
            pl.BlockSpec((512, dout), lambda i: (0, 0)),
            pl.BlockSpec((1, dout), lambda i: (0, 0)),
        ],
        out_specs=pl.BlockSpec((_BLK, dout), lambda i: (i, 0)),
        out_shape=jax.ShapeDtypeStruct((npad, dout), jnp.float32),
        interpret=interpret,
    )(z, wa, ba, wb, bb)


def _l3_pool_body(nblocks, g, z_ref, batch_ref, wa_ref, ba_ref, wb_ref, bb_ref,
                  wl1_ref, bl1_ref, wl2_ref, bl2_ref, out_ref, sums_ref, counts_ref):
    i = pl.program_id(0)

    @pl.when(i == 0)
    def _init():
        sums_ref[...] = jnp.zeros_like(sums_ref)
        counts_ref[...] = jnp.zeros_like(counts_ref)

    z = z_ref[...]
    t = jnp.maximum(
        jnp.dot(z, wa_ref[...], preferred_element_type=jnp.float32) + ba_ref[...], 0.0)
    h3 = jnp.maximum(
        jnp.dot(t, wb_ref[...], preferred_element_type=jnp.float32) + bb_ref[...], 0.0)
    bvec = batch_ref[0]
    gids = lax.broadcasted_iota(jnp.int32, (g, 1), 0)
    onehot = (bvec == gids).astype(jnp.float32)
    sums_ref[...] += jnp.dot(onehot, h3, preferred_element_type=jnp.float32)
    counts_ref[...] += jnp.sum(onehot, axis=1, keepdims=True)

    @pl.when(i == nblocks - 1)
    def _head():
        pooled = sums_ref[...] / jnp.maximum(counts_ref[...], 1.0)
        u = jnp.maximum(
            jnp.dot(pooled, wl1_ref[...], preferred_element_type=jnp.float32)
            + bl1_ref[...], 0.0)
        out_ref[...] = (
            jnp.dot(u, wl2_ref[...], preferred_element_type=jnp.float32) + bl2_ref[...])


def _l3_pool(z, batch3d, wa, ba, wb, bb, wl1, bl1, wl2p, bl2p, g, interpret=False):
    npad = z.shape[0]
    nb = npad // _BLK
    dh = wl2p.shape[1]
    return pl.pallas_call(
        functools.partial(_l3_pool_body, nb, g),
        grid=(nb,),
        in_specs=[
            pl.BlockSpec((_BLK, 512), lambda i: (i, 0)),
            pl.BlockSpec((1, 1, _BLK), lambda i: (i, 0, 0)),
            pl.BlockSpec((512, 512), lambda i: (0, 0)),
            pl.BlockSpec((1, 512), lambda i: (0, 0)),
            pl.BlockSpec((512, 512), lambda i: (0, 0)),
            pl.BlockSpec((1, 512), lambda i: (0, 0)),
            pl.BlockSpec((512, 256), lambda i: (0, 0)),
            pl.BlockSpec((1, 256), lambda i: (0, 0)),
            pl.BlockSpec((256, dh), lambda i: (0, 0)),
            pl.BlockSpec((1, dh), lambda i: (0, 0)),
        ],
        out_specs=pl.BlockSpec((g, dh), lambda i: (0, 0)),
        out_shape=jax.ShapeDtypeStruct((g, dh), jnp.float32),
        scratch_shapes=[
            pltpu.VMEM((g, 512), jnp.float32),
            pltpu.VMEM((g, 1), jnp.float32),
        ],
        interpret=interpret,
    )(z, batch3d, wa, ba, wb, bb, wl1, bl1, wl2p, bl2p)


def _zagg(h, src, dst, n):
    npad = h.shape[0]
    agg = jax.ops.segment_sum(h[src], dst, num_segments=npad)
    return h + agg


def kernel(x, edge_index, batch, w1a, b1a, w1b, b1b, w2a, b2a, w2b, b2b,
           w3a, b3a, w3b, b3b, wl1, bl1, wl2, bl2):
    n = x.shape[0]
    npad = 50176
    src = edge_index[0]
    dst = edge_index[1]

    xp = jnp.zeros((npad, 16), jnp.float32).at[:n, :7].set(x)
    w1a_p = jnp.zeros((16, 512), jnp.float32).at[:7, :].set(w1a)

    b1a_r, b1b_r = b1a[None, :], b1b[None, :]
    b2a_r, b2b_r = b2a[None, :], b2b[None, :]
    b3a_r, b3b_r = b3a[None, :], b3b[None, :]
    bl1_r = bl1[None, :]
    dh = 128
    wl2p = jnp.zeros((256, dh), jnp.float32).at[:, :12].set(wl2)
    bl2p = jnp.zeros((1, dh), jnp.float32).at[0, :12].set(bl2)

    batch3d = jnp.full((npad,), 128, jnp.int32).at[:n].set(batch).reshape(
        npad // _BLK, 1, _BLK)

    z1 = _zagg(xp, src, dst, n)
    h1 = _mlp_layer(z1, w1a_p, b1a_r, w1b, b1b_r)
    z2 = _zagg(h1, src, dst, n)
    h2 = _mlp_layer(z2, w2a, b2a_r, w2b, b2b_r)
    z3 = _zagg(h2, src, dst, n)
    out = _l3_pool(z3, batch3d, w3a, b3a_r, w3b, b3b_r, wl1, bl1_r, wl2p, bl2p, 128)
    return out[:, :12]
